# baseline (speedup 1.0000x reference)
import functools

import jax
import jax.numpy as jnp
from jax import lax
from jax.experimental import pallas as pl
from jax.experimental.pallas import tpu as pltpu
from jax.experimental.pallas import tpu_sc as plsc

_A1 = 0.4
_A2 = 4.4
_P = 6400000
_PQ = _P // 4
_NC = 2
_NS = 16
_NW = _NC * _NS
_L = 16
_PER_W = _PQ // _NW
_CW = 2000
_G = _PER_W // _CW
_BP = 512


def _pack_body(x0, x1, x2, x3, o):
    def code(x):
        return x[0, :] | (x[1, :] << 2)

    o[...] = (code(x0) | (code(x1) << 8) | (code(x2) << 16)
              | (code(x3) << 24))


_pack = pl.pallas_call(
    _pack_body,
    grid=(_PQ // _BP,),
    in_specs=[
        pl.BlockSpec((2, _BP), lambda i, k=k: (0, k * (_PQ // _BP) + i))
        for k in range(4)
    ],
    out_specs=pl.BlockSpec((_BP,), lambda i: (i,)),
    out_shape=jax.ShapeDtypeStruct((_PQ,), jnp.int32),
    compiler_params=pltpu.CompilerParams(
        dimension_semantics=("arbitrary",)),
)


def _body(code_hbm, dist_hbm, cr_hbm, out_hbm, table_v,
          ca, cb, da, db, oa, ob,
          in_sem0, in_sem1, out_sem0, out_sem1):
    wid = lax.axis_index("s") * _NC + lax.axis_index("c")

    pltpu.sync_copy(cr_hbm, table_v)
    t = table_v[...] * _A1 + _A2
    t2 = t * t
    table_v[...] = t2 * t2 * t2

    c_v = (ca, cb)
    d_v = (da, db)
    o_v = (oa, ob)
    in_sems = (in_sem0, in_sem1)
    out_sems = (out_sem0, out_sem1)

    def wbase_of(g):
        return pl.multiple_of(wid * _PER_W + g * _CW, 8)

    def in_copies(g, b):
        wbase = wbase_of(g)
        copies = [(code_hbm.at[pl.ds(wbase, _CW)], c_v[b], in_sems[b])]
        for k in range(4):
            copies.append((
                dist_hbm.at[pl.ds(wbase + k * _PQ, _CW)],
                d_v[b].at[pl.ds(k * _CW, _CW)],
                in_sems[b],
            ))
        return copies

    def out_copies(g, b):
        wbase = wbase_of(g)
        return [
            (o_v[b].at[pl.ds(k * _CW, _CW)],
             out_hbm.at[pl.ds(wbase + k * _PQ, _CW)],
             out_sems[b])
            for k in range(4)
        ]

    def start_in(g, b):
        for args in in_copies(g, b):
            pltpu.async_copy(*args)

    def wait_in(g, b):
        for args in in_copies(g, b):
            pltpu.make_async_copy(*args).wait()

    def start_out(g, b):
        for args in out_copies(g, b):
            pltpu.async_copy(*args)

    def wait_out(g, b):
        for args in out_copies(g, b):
            pltpu.make_async_copy(*args).wait()

    def compute(b):
        @plsc.parallel_loop(0, _CW, _L, unroll=4)
        def inner(w):
            c32 = c_v[b][pl.ds(w, _L)]
            for k in range(4):
                idx = (c32 >> (8 * k)) & 0xF
                damp = plsc.load_gather(table_v, [idx])
                d = d_v[b][pl.ds(k * _CW + w, _L)]
                d2 = d * d
                o_v[b][pl.ds(k * _CW + w, _L)] = d2 * d2 * d2 + damp

    start_in(0, 0)
    start_in(1, 1)
    for j in (0, 1):
        wait_in(j, j)
        compute(j)
        start_out(j, j)
        start_in(2 + j, j)

    def round_body(gp, carry):
        for j in (0, 1):
            g = gp * 2 + j
            wait_in(g, j)
            wait_out(g - 2, j)
            compute(j)
            start_out(g, j)
            start_in(g + 2, j)
        return carry

    lax.fori_loop(1, _G // 2 - 1, round_body, 0)

    for j in (0, 1):
        g = _G - 3 + j
        wait_in(g, j)
        wait_out(g - 2, j)
        compute(j)
        start_out(g, j)
        if g + 2 <= _G - 1:
            start_in(g + 2, j)

    g = _G - 1
    wait_in(g, 0)
    wait_out(g - 2, 0)
    compute(0)
    start_out(g, 0)

    wait_out(_G - 2, 1)
    wait_out(_G - 1, 0)


_damp = functools.partial(
    pl.kernel,
    out_type=jax.ShapeDtypeStruct((_P,), jnp.float32),
    mesh=plsc.VectorSubcoreMesh(core_axis_name="c", subcore_axis_name="s"),
    scratch_types=[
        pltpu.VMEM((16,), jnp.float32),
        pltpu.VMEM((_CW,), jnp.int32),
        pltpu.VMEM((_CW,), jnp.int32),
        pltpu.VMEM((4 * _CW,), jnp.float32),
        pltpu.VMEM((4 * _CW,), jnp.float32),
        pltpu.VMEM((4 * _CW,), jnp.float32),
        pltpu.VMEM((4 * _CW,), jnp.float32),
        pltpu.SemaphoreType.DMA,
        pltpu.SemaphoreType.DMA,
        pltpu.SemaphoreType.DMA,
        pltpu.SemaphoreType.DMA,
    ],
    compiler_params=pltpu.CompilerParams(needs_layout_passes=False),
)(_body)


@jax.jit
def kernel(species12, distances, cutoff_radii):
    codes = _pack(species12, species12, species12, species12)
    return _damp(codes, distances, cutoff_radii.T.reshape(-1))


# speedup vs baseline: 12.3839x; 12.3839x over previous
import functools

import jax
import jax.numpy as jnp
from jax import lax
from jax.experimental import pallas as pl
from jax.experimental.pallas import tpu as pltpu
from jax.experimental.pallas import tpu_sc as plsc

_A1 = 0.4
_A2 = 4.4
_P = 6400000
_PQ = _P // 4
_NC = 2
_NS = 16
_NW = _NC * _NS
_L = 16
_PER_W = _PQ // _NW
_CW = 2000
_G = _PER_W // _CW
_BP = 12800


def _pack_body(x0, x1, x2, x3, o):
    def code(x):
        return x[0, :] | (x[1, :] << 2)

    i = pl.program_id(0)
    o[pl.ds(i * _BP, _BP)] = (code(x0) | (code(x1) << 8)
                              | (code(x2) << 16) | (code(x3) << 24))


_pack = pl.pallas_call(
    _pack_body,
    grid=(_PQ // _BP,),
    in_specs=[
        pl.BlockSpec((2, _BP), lambda i, k=k: (0, k * (_PQ // _BP) + i))
        for k in range(4)
    ],
    out_specs=pl.BlockSpec((_PQ,), lambda i: (0,)),
    out_shape=jax.ShapeDtypeStruct((_PQ,), jnp.int32),
    compiler_params=pltpu.CompilerParams(
        dimension_semantics=("arbitrary",)),
)


def _body(code_hbm, dist_hbm, cr_hbm, out_hbm, table_v,
          ca, cb, da, db, oa, ob,
          in_sem0, in_sem1, out_sem0, out_sem1):
    wid = lax.axis_index("s") * _NC + lax.axis_index("c")

    pltpu.sync_copy(cr_hbm, table_v)
    t = table_v[...] * _A1 + _A2
    t2 = t * t
    table_v[...] = t2 * t2 * t2

    c_v = (ca, cb)
    d_v = (da, db)
    o_v = (oa, ob)
    in_sems = (in_sem0, in_sem1)
    out_sems = (out_sem0, out_sem1)

    def wbase_of(g):
        return pl.multiple_of(wid * _PER_W + g * _CW, 8)

    def in_copies(g, b):
        wbase = wbase_of(g)
        copies = [(code_hbm.at[pl.ds(wbase, _CW)], c_v[b], in_sems[b])]
        for k in range(4):
            copies.append((
                dist_hbm.at[pl.ds(wbase + k * _PQ, _CW)],
                d_v[b].at[pl.ds(k * _CW, _CW)],
                in_sems[b],
            ))
        return copies

    def out_copies(g, b):
        wbase = wbase_of(g)
        return [
            (o_v[b].at[pl.ds(k * _CW, _CW)],
             out_hbm.at[pl.ds(wbase + k * _PQ, _CW)],
             out_sems[b])
            for k in range(4)
        ]

    def start_in(g, b):
        for args in in_copies(g, b):
            pltpu.async_copy(*args)

    def wait_in(g, b):
        for args in in_copies(g, b):
            pltpu.make_async_copy(*args).wait()

    def start_out(g, b):
        for args in out_copies(g, b):
            pltpu.async_copy(*args)

    def wait_out(g, b):
        for args in out_copies(g, b):
            pltpu.make_async_copy(*args).wait()

    def compute(b):
        @plsc.parallel_loop(0, _CW, _L, unroll=4)
        def inner(w):
            c32 = c_v[b][pl.ds(w, _L)]
            for k in range(4):
                idx = (c32 >> (8 * k)) & 0xF
                damp = plsc.load_gather(table_v, [idx])
                d = d_v[b][pl.ds(k * _CW + w, _L)]
                d2 = d * d
                o_v[b][pl.ds(k * _CW + w, _L)] = d2 * d2 * d2 + damp

    start_in(0, 0)
    start_in(1, 1)
    for j in (0, 1):
        wait_in(j, j)
        compute(j)
        start_out(j, j)
        start_in(2 + j, j)

    def round_body(gp, carry):
        for j in (0, 1):
            g = gp * 2 + j
            wait_in(g, j)
            wait_out(g - 2, j)
            compute(j)
            start_out(g, j)
            start_in(g + 2, j)
        return carry

    lax.fori_loop(1, _G // 2 - 1, round_body, 0)

    for j in (0, 1):
        g = _G - 3 + j
        wait_in(g, j)
        wait_out(g - 2, j)
        compute(j)
        start_out(g, j)
        if g + 2 <= _G - 1:
            start_in(g + 2, j)

    g = _G - 1
    wait_in(g, 0)
    wait_out(g - 2, 0)
    compute(0)
    start_out(g, 0)

    wait_out(_G - 2, 1)
    wait_out(_G - 1, 0)


_damp = functools.partial(
    pl.kernel,
    out_type=jax.ShapeDtypeStruct((_P,), jnp.float32),
    mesh=plsc.VectorSubcoreMesh(core_axis_name="c", subcore_axis_name="s"),
    scratch_types=[
        pltpu.VMEM((16,), jnp.float32),
        pltpu.VMEM((_CW,), jnp.int32),
        pltpu.VMEM((_CW,), jnp.int32),
        pltpu.VMEM((4 * _CW,), jnp.float32),
        pltpu.VMEM((4 * _CW,), jnp.float32),
        pltpu.VMEM((4 * _CW,), jnp.float32),
        pltpu.VMEM((4 * _CW,), jnp.float32),
        pltpu.SemaphoreType.DMA,
        pltpu.SemaphoreType.DMA,
        pltpu.SemaphoreType.DMA,
        pltpu.SemaphoreType.DMA,
    ],
    compiler_params=pltpu.CompilerParams(needs_layout_passes=False),
)(_body)


@jax.jit
def kernel(species12, distances, cutoff_radii):
    codes = _pack(species12, species12, species12, species12)
    return _damp(codes, distances, cutoff_radii.T.reshape(-1))


# speedup vs baseline: 20.0088x; 1.6157x over previous
import functools

import jax
import jax.numpy as jnp
from jax import lax
from jax.experimental import pallas as pl
from jax.experimental.pallas import tpu as pltpu
from jax.experimental.pallas import tpu_sc as plsc

_A1 = 0.4
_A2 = 4.4
_P = 6400000
_PQ = _P // 4
_NC = 2
_NS = 16
_NW = _NC * _NS
_L = 16
_PER_W = _PQ // _NW
_CW = 2000
_G = _PER_W // _CW
_BP = 64000


def _pack_body(x0, x1, x2, x3, o):
    def code(x):
        return x[0, :] | (x[1, :] << 2)

    i = pl.program_id(0)
    o[pl.ds(i * _BP, _BP)] = (code(x0) | (code(x1) << 8)
                              | (code(x2) << 16) | (code(x3) << 24))


_pack = pl.pallas_call(
    _pack_body,
    grid=(_PQ // _BP,),
    in_specs=[
        pl.BlockSpec((2, _BP), lambda i, k=k: (0, k * (_PQ // _BP) + i))
        for k in range(4)
    ],
    out_specs=pl.BlockSpec((_PQ,), lambda i: (0,)),
    out_shape=jax.ShapeDtypeStruct((_PQ,), jnp.int32),
    compiler_params=pltpu.CompilerParams(
        dimension_semantics=("arbitrary",)),
)


def _body(code_hbm, dist_hbm, cr_hbm, out_hbm, table_v,
          ca, cb, da, db, oa, ob,
          in_sem0, in_sem1, out_sem0, out_sem1):
    wid = lax.axis_index("s") * _NC + lax.axis_index("c")

    pltpu.sync_copy(cr_hbm, table_v)
    t = table_v[...] * _A1 + _A2
    t2 = t * t
    table_v[...] = t2 * t2 * t2

    c_v = (ca, cb)
    d_v = (da, db)
    o_v = (oa, ob)
    in_sems = (in_sem0, in_sem1)
    out_sems = (out_sem0, out_sem1)

    def wbase_of(g):
        return pl.multiple_of(wid * _PER_W + g * _CW, 8)

    def in_copies(g, b):
        wbase = wbase_of(g)
        copies = [(code_hbm.at[pl.ds(wbase, _CW)], c_v[b], in_sems[b])]
        for k in range(4):
            copies.append((
                dist_hbm.at[pl.ds(wbase + k * _PQ, _CW)],
                d_v[b].at[pl.ds(k * _CW, _CW)],
                in_sems[b],
            ))
        return copies

    def out_copies(g, b):
        wbase = wbase_of(g)
        return [
            (o_v[b].at[pl.ds(k * _CW, _CW)],
             out_hbm.at[pl.ds(wbase + k * _PQ, _CW)],
             out_sems[b])
            for k in range(4)
        ]

    def start_in(g, b):
        for args in in_copies(g, b):
            pltpu.async_copy(*args)

    def wait_in(g, b):
        for args in in_copies(g, b):
            pltpu.make_async_copy(*args).wait()

    def start_out(g, b):
        for args in out_copies(g, b):
            pltpu.async_copy(*args)

    def wait_out(g, b):
        for args in out_copies(g, b):
            pltpu.make_async_copy(*args).wait()

    def compute(b):
        @plsc.parallel_loop(0, _CW, _L, unroll=4)
        def inner(w):
            c32 = c_v[b][pl.ds(w, _L)]
            for k in range(4):
                idx = (c32 >> (8 * k)) & 0xF
                damp = plsc.load_gather(table_v, [idx])
                d = d_v[b][pl.ds(k * _CW + w, _L)]
                d2 = d * d
                o_v[b][pl.ds(k * _CW + w, _L)] = d2 * d2 * d2 + damp

    start_in(0, 0)
    start_in(1, 1)
    for j in (0, 1):
        wait_in(j, j)
        compute(j)
        start_out(j, j)
        start_in(2 + j, j)

    def round_body(gp, carry):
        for j in (0, 1):
            g = gp * 2 + j
            wait_in(g, j)
            wait_out(g - 2, j)
            compute(j)
            start_out(g, j)
            start_in(g + 2, j)
        return carry

    lax.fori_loop(1, _G // 2 - 1, round_body, 0)

    for j in (0, 1):
        g = _G - 3 + j
        wait_in(g, j)
        wait_out(g - 2, j)
        compute(j)
        start_out(g, j)
        if g + 2 <= _G - 1:
            start_in(g + 2, j)

    g = _G - 1
    wait_in(g, 0)
    wait_out(g - 2, 0)
    compute(0)
    start_out(g, 0)

    wait_out(_G - 2, 1)
    wait_out(_G - 1, 0)


_damp = functools.partial(
    pl.kernel,
    out_type=jax.ShapeDtypeStruct((_P,), jnp.float32),
    mesh=plsc.VectorSubcoreMesh(core_axis_name="c", subcore_axis_name="s"),
    scratch_types=[
        pltpu.VMEM((16,), jnp.float32),
        pltpu.VMEM((_CW,), jnp.int32),
        pltpu.VMEM((_CW,), jnp.int32),
        pltpu.VMEM((4 * _CW,), jnp.float32),
        pltpu.VMEM((4 * _CW,), jnp.float32),
        pltpu.VMEM((4 * _CW,), jnp.float32),
        pltpu.VMEM((4 * _CW,), jnp.float32),
        pltpu.SemaphoreType.DMA,
        pltpu.SemaphoreType.DMA,
        pltpu.SemaphoreType.DMA,
        pltpu.SemaphoreType.DMA,
    ],
    compiler_params=pltpu.CompilerParams(needs_layout_passes=False),
)(_body)


@jax.jit
def kernel(species12, distances, cutoff_radii):
    codes = _pack(species12, species12, species12, species12)
    return _damp(codes, distances, cutoff_radii.T.reshape(-1))
